# Initial kernel scaffold; baseline (speedup 1.0000x reference)
import jax
import jax.numpy as jnp
from jax.experimental import pallas as pl


def kernel(x, W):
    raise NotImplementedError("write your pallas kernel here")


# speedup vs baseline: 4.6855x; 4.6855x over previous
import jax
import jax.numpy as jnp
from jax.experimental import pallas as pl
from jax.experimental.pallas import tpu as pltpu

_TOPK = 8
_E = 64
_D = 4096


def _router_block(x_ref, wt_ref, idx_ref, tkw_ref, full_ref):
    xb = x_ref[...]
    logits = jnp.dot(xb, wt_ref[...], preferred_element_type=jnp.float32)
    bt = logits.shape[0]
    eidx = jax.lax.broadcasted_iota(jnp.int32, (bt, _E), 1)
    work = logits
    neg = jnp.float32(-jnp.inf)
    vals, idxs, hots = [], [], []
    for _ in range(_TOPK):
        m = jnp.max(work, axis=1, keepdims=True)
        first = jnp.min(jnp.where(work == m, eidx, _E), axis=1, keepdims=True)
        hot = eidx == first
        vals.append(m)
        idxs.append(first)
        hots.append(hot)
        work = jnp.where(hot, neg, work)
    exps = [jnp.exp(v - vals[0]) for v in vals]
    tot = exps[0]
    for e in exps[1:]:
        tot = tot + e
    inv = 1.0 / tot
    ws = [e * inv for e in exps]
    idx_ref[...] = jnp.concatenate(idxs, axis=1).astype(jnp.int32)
    tkw_ref[...] = jnp.concatenate(ws, axis=1)
    full = jnp.where(hots[0], ws[0], jnp.float32(0.0))
    for k in range(1, _TOPK):
        full = full + jnp.where(hots[k], ws[k], jnp.float32(0.0))
    full_ref[...] = full


def kernel(x, W):
    tokens = x.shape[0]
    bt = 512
    wt = W.T
    out_shape = (
        jax.ShapeDtypeStruct((tokens, _TOPK), jnp.int32),
        jax.ShapeDtypeStruct((tokens, _TOPK), jnp.float32),
        jax.ShapeDtypeStruct((tokens, _E), jnp.float32),
    )
    return pl.pallas_call(
        _router_block,
        grid=(tokens // bt,),
        in_specs=[
            pl.BlockSpec((bt, _D), lambda i: (i, 0)),
            pl.BlockSpec((_D, _E), lambda i: (0, 0)),
        ],
        out_specs=(
            pl.BlockSpec((bt, _TOPK), lambda i: (i, 0)),
            pl.BlockSpec((bt, _TOPK), lambda i: (i, 0)),
            pl.BlockSpec((bt, _E), lambda i: (i, 0)),
        ),
        out_shape=out_shape,
    )(x, wt)


# speedup vs baseline: 5.3598x; 1.1439x over previous
import jax
import jax.numpy as jnp
from jax.experimental import pallas as pl
from jax.experimental.pallas import tpu as pltpu

_TOPK = 8
_E = 64
_D = 4096


def _router_block(x_ref, wt_ref, idx_ref, tkw_ref, full_ref):
    xb = x_ref[...]
    logits = jnp.dot(xb, wt_ref[...], preferred_element_type=jnp.float32)
    bt = logits.shape[0]
    eidx = jax.lax.broadcasted_iota(jnp.int32, (bt, _E), 1)
    work = logits
    neg = jnp.float32(-jnp.inf)
    vals, idxs, hots = [], [], []
    for _ in range(_TOPK):
        m = jnp.max(work, axis=1, keepdims=True)
        first = jnp.min(jnp.where(work == m, eidx, _E), axis=1, keepdims=True)
        hot = eidx == first
        vals.append(m)
        idxs.append(first)
        hots.append(hot)
        work = jnp.where(hot, neg, work)
    exps = [jnp.exp(v - vals[0]) for v in vals]
    tot = exps[0]
    for e in exps[1:]:
        tot = tot + e
    inv = 1.0 / tot
    ws = [e * inv for e in exps]
    idx_ref[...] = jnp.concatenate(idxs, axis=1).astype(jnp.int32)
    tkw_ref[...] = jnp.concatenate(ws, axis=1)
    full = jnp.where(hots[0], ws[0], jnp.float32(0.0))
    for k in range(1, _TOPK):
        full = full + jnp.where(hots[k], ws[k], jnp.float32(0.0))
    full_ref[...] = full


def kernel(x, W):
    tokens = x.shape[0]
    bt = 1024
    wt = W.T
    out_shape = (
        jax.ShapeDtypeStruct((tokens, _TOPK), jnp.int32),
        jax.ShapeDtypeStruct((tokens, _TOPK), jnp.float32),
        jax.ShapeDtypeStruct((tokens, _E), jnp.float32),
    )
    return pl.pallas_call(
        _router_block,
        grid=(tokens // bt,),
        in_specs=[
            pl.BlockSpec((bt, _D), lambda i: (i, 0)),
            pl.BlockSpec((_D, _E), lambda i: (0, 0)),
        ],
        out_specs=(
            pl.BlockSpec((bt, _TOPK), lambda i: (i, 0)),
            pl.BlockSpec((bt, _TOPK), lambda i: (i, 0)),
            pl.BlockSpec((bt, _E), lambda i: (i, 0)),
        ),
        out_shape=out_shape,
    )(x, wt)
